# baseline (speedup 1.0000x reference)
import jax
import jax.numpy as jnp
from jax.experimental import pallas as pl
from jax.experimental.pallas import tpu as pltpu

B = 1024
D = 1024
E = 8
T = 12
BME = 512


def _enc_body(x_ref, w1_ref, b1_ref, w2_ref, b2_ref, out_ref):
    x = x_ref[0]
    h = jnp.dot(x, w1_ref[0], preferred_element_type=jnp.float32) + b1_ref[0, 0]
    h = jnp.maximum(h, 0.0)
    o = jnp.dot(h, w2_ref[0], preferred_element_type=jnp.float32) + b2_ref[0, 0]
    n = jnp.sum(jnp.abs(o), axis=1, keepdims=True)
    out_ref[0] = o / jnp.maximum(n, 1e-12)


def _router_body(x_ref, wg_ref, combine_ref, aux_ref):
    x = x_ref[0]
    logits = jnp.dot(x, wg_ref[0], preferred_element_type=jnp.float32)
    m = jnp.max(logits, axis=-1, keepdims=True)
    ex = jnp.exp(logits - m)
    probs = ex / jnp.sum(ex, axis=-1, keepdims=True)
    tri = (jax.lax.broadcasted_iota(jnp.int32, (E, E), 0)
           <= jax.lax.broadcasted_iota(jnp.int32, (E, E), 1)).astype(jnp.float32)
    m1 = jnp.max(probs, axis=-1, keepdims=True)
    is1 = probs == m1
    cs1 = jnp.dot(is1.astype(jnp.float32), tri, preferred_element_type=jnp.float32)
    first1 = jnp.logical_and(is1, cs1 == 1.0)
    p2 = jnp.where(first1, -1.0, probs)
    m2 = jnp.max(p2, axis=-1, keepdims=True)
    is2 = p2 == m2
    cs2 = jnp.dot(is2.astype(jnp.float32), tri, preferred_element_type=jnp.float32)
    first2 = jnp.logical_and(is2, cs2 == 1.0)
    gsum = m1 + m2
    combine = (first1.astype(jnp.float32) * (m1 / gsum)
               + first2.astype(jnp.float32) * (m2 / gsum))
    combine_ref[0] = combine.T
    fe = jnp.mean(first1.astype(jnp.float32), axis=0)
    pe = jnp.mean(probs, axis=0)
    aux_ref[0, 0] = fe * pe


def _moe_body(x_ref, w1_ref, b1_ref, w2_ref, b2_ref, c_ref, out_ref):
    e = pl.program_id(1)
    x = x_ref[0]
    h = jnp.dot(x, w1_ref[0, 0], preferred_element_type=jnp.float32) + b1_ref[0, 0, 0]
    h = jnp.maximum(h, 0.0)
    o = jnp.dot(h, w2_ref[0, 0], preferred_element_type=jnp.float32) + b2_ref[0, 0, 0]
    val = o * c_ref[0, 0, 0][:, None]

    @pl.when(e == 0)
    def _():
        out_ref[0] = val

    @pl.when(e != 0)
    def _():
        out_ref[0] = out_ref[0] + val


def kernel(input_molecule, params):
    Xv = jnp.transpose(input_molecule, (1, 0, 2))[jnp.array([1, 2, 3, 0])]

    enc_order = (1, 2, 3, 0)
    w1e = jnp.stack([params['enc'][i]['w1'] for i in enc_order])
    b1e = jnp.stack([params['enc'][i]['b1'] for i in enc_order])[:, None, :]
    w2e = jnp.stack([params['enc'][i]['w2'] for i in enc_order])
    b2e = jnp.stack([params['enc'][i]['b2'] for i in enc_order])[:, None, :]

    wg = jnp.stack([p['wg'] for p in params['clf']])
    w1c = jnp.stack([p['w1'] for p in params['clf']])
    b1c = jnp.stack([p['b1'] for p in params['clf']])[:, :, None, :]
    w2c = jnp.stack([p['w2'] for p in params['clf']])
    b2c = jnp.stack([p['b2'] for p in params['clf']])[:, :, None, :]

    enc_out = pl.pallas_call(
        _enc_body,
        grid=(4, B // BME),
        in_specs=[
            pl.BlockSpec((1, BME, D), lambda v, m: (v, m, 0)),
            pl.BlockSpec((1, D, D), lambda v, m: (v, 0, 0)),
            pl.BlockSpec((1, 1, D), lambda v, m: (v, 0, 0)),
            pl.BlockSpec((1, D, D), lambda v, m: (v, 0, 0)),
            pl.BlockSpec((1, 1, D), lambda v, m: (v, 0, 0)),
        ],
        out_specs=pl.BlockSpec((1, BME, D), lambda v, m: (v, m, 0)),
        out_shape=jax.ShapeDtypeStruct((4, B, D), jnp.float32),
    )(Xv, w1e, b1e, w2e, b2e)

    combine_t, aux = pl.pallas_call(
        _router_body,
        grid=(4,),
        in_specs=[
            pl.BlockSpec((1, B, D), lambda v: (v, 0, 0)),
            pl.BlockSpec((1, D, E), lambda v: (v, 0, 0)),
        ],
        out_specs=[
            pl.BlockSpec((1, E, B), lambda v: (v, 0, 0)),
            pl.BlockSpec((1, 1, E), lambda v: (v, 0, 0)),
        ],
        out_shape=[
            jax.ShapeDtypeStruct((4, E, B), jnp.float32),
            jax.ShapeDtypeStruct((4, 1, E), jnp.float32),
        ],
    )(Xv, wg)

    moe_out = pl.pallas_call(
        _moe_body,
        grid=(4, E),
        in_specs=[
            pl.BlockSpec((1, B, D), lambda v, e: (v, 0, 0)),
            pl.BlockSpec((1, 1, D, D), lambda v, e: (v, e, 0, 0)),
            pl.BlockSpec((1, 1, 1, D), lambda v, e: (v, e, 0, 0)),
            pl.BlockSpec((1, 1, D, T), lambda v, e: (v, e, 0, 0)),
            pl.BlockSpec((1, 1, 1, T), lambda v, e: (v, e, 0, 0)),
            pl.BlockSpec((1, 1, 1, B), lambda v, e: (v, e, 0, 0)),
        ],
        out_specs=pl.BlockSpec((1, B, T), lambda v, e: (v, 0, 0)),
        out_shape=jax.ShapeDtypeStruct((4, B, T), jnp.float32),
    )(Xv, w1c, b1c, w2c, b2c, combine_t.reshape(4, E, 1, B))

    loss_auc = E * jnp.sum(aux)
    return (moe_out[3], moe_out[0], moe_out[1], moe_out[2],
            enc_out[0], enc_out[1], enc_out[2], enc_out[3], loss_auc)
